# Initial kernel scaffold; baseline (speedup 1.0000x reference)
import jax
import jax.numpy as jnp
from jax.experimental import pallas as pl


def kernel(x, edge_index, W0, b0, W1, b1, W2, b2, W_fc, b_fc):
    raise NotImplementedError("write your pallas kernel here")


# speedup vs baseline: 2.7418x; 2.7418x over previous
import functools

import jax
import jax.numpy as jnp
from jax import lax
from jax.experimental import pallas as pl
from jax.experimental.pallas import tpu as pltpu
from jax.experimental.pallas import tpu_sc as plsc

N = 10000
D = 256
E = 160000
OUT = 64
H = 128
NS = 16
BE = 128
NB_ALL = E // BE
NB_LO = NB_ALL // NS
NB_EXTRA = NB_ALL - NB_LO * NS
ROWS_PER_TILE = 624
ROW_TAIL = N - ROWS_PER_TILE * NS

_mesh = plsc.VectorSubcoreMesh(
    core_axis_name="c", subcore_axis_name="s", num_cores=2, num_subcores=NS
)


def _leaky(v):
    return jnp.where(v >= 0, v, 0.01 * v)


@functools.partial(
    pl.kernel,
    out_type=[
        jax.ShapeDtypeStruct((N,), jnp.float32),
        jax.ShapeDtypeStruct((N,), jnp.float32),
    ],
    mesh=_mesh,
    scratch_types=[
        pltpu.VMEM((BE,), jnp.int32),
        pltpu.VMEM((BE,), jnp.float32),
        pltpu.VMEM_SHARED((N,), jnp.float32),
    ],
)
def _deg(src_hbm, dst_hbm, zer_hbm, outs_hbm, outd_hbm, idx_v, ones_v, acc_sh):
    cid = lax.axis_index("c")
    sid = lax.axis_index("s")

    def fill_ones(i, c):
        ones_v[pl.ds(i * 16, 16)] = jnp.ones((16,), jnp.float32)
        return c

    lax.fori_loop(0, BE // 16, fill_ones, 0)

    @pl.when(sid == 0)
    def _zero():
        pltpu.sync_copy(zer_hbm, acc_sh)

    plsc.subcore_barrier()

    nb = jnp.where(sid < NB_EXTRA, NB_LO + 1, NB_LO)

    def run(edges_hbm):
        def body(i, c):
            off = (sid + NS * i) * BE
            pltpu.sync_copy(edges_hbm.at[pl.ds(off, BE)], idx_v)
            pltpu.sync_copy(ones_v, acc_sh.at[idx_v], add=True)
            return c

        lax.fori_loop(0, nb, body, 0)

    @pl.when(cid == 0)
    def _src():
        run(src_hbm)

    @pl.when(cid == 1)
    def _dst():
        run(dst_hbm)

    plsc.subcore_barrier()

    @pl.when((sid == 0) & (cid == 0))
    def _ws():
        pltpu.sync_copy(acc_sh, outs_hbm)

    @pl.when((sid == 0) & (cid == 1))
    def _wd():
        pltpu.sync_copy(acc_sh, outd_hbm)


@functools.partial(
    pl.kernel,
    out_type=[
        jax.ShapeDtypeStruct((N, H), jnp.float32),
        jax.ShapeDtypeStruct((N, H), jnp.float32),
    ],
    mesh=_mesh,
    scratch_types=[
        pltpu.VMEM((BE,), jnp.int32),
        pltpu.VMEM((BE,), jnp.int32),
        pltpu.VMEM((BE, H), jnp.float32),
        pltpu.VMEM_SHARED((N, H), jnp.float32),
        pltpu.SemaphoreType.DMA,
    ],
)
def _msg(t0_hbm, t1_hbm, src_hbm, dst_hbm, zrow_hbm, out0_hbm, out1_hbm,
         sidx_v, didx_v, rows_v, acc_sh, sem):
    cid = lax.axis_index("c")
    sid = lax.axis_index("s")

    pltpu.sync_copy(zrow_hbm, acc_sh.at[pl.ds(sid * ROWS_PER_TILE, ROWS_PER_TILE)])

    @pl.when(sid == NS - 1)
    def _ztail():
        pltpu.sync_copy(zrow_hbm.at[pl.ds(0, ROW_TAIL)],
                        acc_sh.at[pl.ds(NS * ROWS_PER_TILE, ROW_TAIL)])

    plsc.subcore_barrier()

    nb = jnp.where(sid < NB_EXTRA, NB_LO + 1, NB_LO)

    def run(t_hbm):
        def body(i, c):
            off = (sid + NS * i) * BE
            pltpu.sync_copy(src_hbm.at[pl.ds(off, BE)], sidx_v)
            pltpu.sync_copy(dst_hbm.at[pl.ds(off, BE)], didx_v)
            pltpu.async_copy(t_hbm.at[sidx_v], rows_v, sem).wait()
            pltpu.sync_copy(rows_v, acc_sh.at[didx_v], add=True)
            return c

        lax.fori_loop(0, nb, body, 0)

    @pl.when(cid == 0)
    def _h0():
        run(t0_hbm)

    @pl.when(cid == 1)
    def _h1():
        run(t1_hbm)

    plsc.subcore_barrier()

    rsl = pl.ds(sid * ROWS_PER_TILE, ROWS_PER_TILE)
    tsl = pl.ds(NS * ROWS_PER_TILE, ROW_TAIL)

    @pl.when(cid == 0)
    def _w0():
        pltpu.sync_copy(acc_sh.at[rsl], out0_hbm.at[rsl])

    @pl.when(cid == 1)
    def _w1():
        pltpu.sync_copy(acc_sh.at[rsl], out1_hbm.at[rsl])

    @pl.when((sid == NS - 1) & (cid == 0))
    def _w0t():
        pltpu.sync_copy(acc_sh.at[tsl], out0_hbm.at[tsl])

    @pl.when((sid == NS - 1) & (cid == 1))
    def _w1t():
        pltpu.sync_copy(acc_sh.at[tsl], out1_hbm.at[tsl])


BM = 400


def _mm0_body(x_ref, w_ref, degs_ref, o0_ref, o1_ref):
    t = jnp.dot(x_ref[...], w_ref[...], preferred_element_type=jnp.float32)
    t = t * lax.rsqrt(degs_ref[...] + 1.0)
    o0_ref[...] = t[:, :H]
    o1_ref[...] = t[:, H:]


_mm0 = pl.pallas_call(
    _mm0_body,
    grid=(N // BM,),
    in_specs=[
        pl.BlockSpec((BM, D), lambda i: (i, 0)),
        pl.BlockSpec((D, D), lambda i: (0, 0)),
        pl.BlockSpec((BM, 1), lambda i: (i, 0)),
    ],
    out_specs=[
        pl.BlockSpec((BM, H), lambda i: (i, 0)),
        pl.BlockSpec((BM, H), lambda i: (i, 0)),
    ],
    out_shape=[
        jax.ShapeDtypeStruct((N, H), jnp.float32),
        jax.ShapeDtypeStruct((N, H), jnp.float32),
    ],
)


def _mm_body(a0_ref, a1_ref, p0_ref, p1_ref, degd_ref, b_ref, w_ref,
             degs_ref, o0_ref, o1_ref):
    agg = jnp.concatenate([a0_ref[...] + p0_ref[...],
                           a1_ref[...] + p1_ref[...]], axis=1)
    h = _leaky(agg * lax.rsqrt(degd_ref[...] + 1.0) + b_ref[...])
    t = jnp.dot(h, w_ref[...], preferred_element_type=jnp.float32)
    t = t * lax.rsqrt(degs_ref[...] + 1.0)
    o0_ref[...] = t[:, :H]
    o1_ref[...] = t[:, H:]


_mm = pl.pallas_call(
    _mm_body,
    grid=(N // BM,),
    in_specs=[
        pl.BlockSpec((BM, H), lambda i: (i, 0)),
        pl.BlockSpec((BM, H), lambda i: (i, 0)),
        pl.BlockSpec((BM, H), lambda i: (i, 0)),
        pl.BlockSpec((BM, H), lambda i: (i, 0)),
        pl.BlockSpec((BM, 1), lambda i: (i, 0)),
        pl.BlockSpec((1, D), lambda i: (0, 0)),
        pl.BlockSpec((D, D), lambda i: (0, 0)),
        pl.BlockSpec((BM, 1), lambda i: (i, 0)),
    ],
    out_specs=[
        pl.BlockSpec((BM, H), lambda i: (i, 0)),
        pl.BlockSpec((BM, H), lambda i: (i, 0)),
    ],
    out_shape=[
        jax.ShapeDtypeStruct((N, H), jnp.float32),
        jax.ShapeDtypeStruct((N, H), jnp.float32),
    ],
)


BN = 40
FC_STEPS = N // BN


def _fc_body(a0_ref, a1_ref, p0_ref, p1_ref, degd_ref, b_ref, wfc_ref,
             bfc_ref, out_ref, acc_s):
    i = pl.program_id(0)

    @pl.when(i == 0)
    def _init():
        acc_s[...] = jnp.zeros((OUT, D), jnp.float32)

    agg = jnp.concatenate([a0_ref[...] + p0_ref[...],
                           a1_ref[...] + p1_ref[...]], axis=1)
    h = _leaky(agg * lax.rsqrt(degd_ref[...] + 1.0) + b_ref[...])
    f = _leaky(h)
    w = wfc_ref[...]
    acc = acc_s[...]
    for n in range(BN):
        acc = acc + w[:, n, :] * f[n][None, :]
    acc_s[...] = acc

    @pl.when(i == FC_STEPS - 1)
    def _fin():
        out_ref[...] = jnp.sum(acc_s[...], axis=1) + bfc_ref[...]


_fc = pl.pallas_call(
    _fc_body,
    grid=(FC_STEPS,),
    in_specs=[
        pl.BlockSpec((BN, H), lambda i: (i, 0)),
        pl.BlockSpec((BN, H), lambda i: (i, 0)),
        pl.BlockSpec((BN, H), lambda i: (i, 0)),
        pl.BlockSpec((BN, H), lambda i: (i, 0)),
        pl.BlockSpec((BN, 1), lambda i: (i, 0)),
        pl.BlockSpec((1, D), lambda i: (0, 0)),
        pl.BlockSpec((OUT, BN, D), lambda i: (0, i, 0)),
        pl.BlockSpec((OUT,), lambda i: (0,)),
    ],
    out_specs=pl.BlockSpec((OUT,), lambda i: (0,)),
    out_shape=jax.ShapeDtypeStruct((OUT,), jnp.float32),
    scratch_shapes=[pltpu.VMEM((OUT, D), jnp.float32)],
)


def kernel(x, edge_index, W0, b0, W1, b1, W2, b2, W_fc, b_fc):
    src = edge_index[0]
    dst = edge_index[1]
    zer_deg = jnp.zeros((N,), jnp.float32)
    zer_row = jnp.zeros((ROWS_PER_TILE, H), jnp.float32)

    degs, degd = _deg(src, dst, zer_deg)
    degs = degs.reshape(N, 1)
    degd = degd.reshape(N, 1)

    t0, t1 = _mm0(x, W0, degs)
    a0, a1 = _msg(t0, t1, src, dst, zer_row)
    t0, t1 = _mm(a0, a1, t0, t1, degd, b0.reshape(1, D), W1, degs)
    a0, a1 = _msg(t0, t1, src, dst, zer_row)
    t0, t1 = _mm(a0, a1, t0, t1, degd, b1.reshape(1, D), W2, degs)
    a0, a1 = _msg(t0, t1, src, dst, zer_row)
    out = _fc(a0, a1, t0, t1, degd, b2.reshape(1, D),
              W_fc.reshape(OUT, N, D), b_fc)
    return out


# speedup vs baseline: 6.9684x; 2.5416x over previous
import functools

import jax
import jax.numpy as jnp
from jax import lax
from jax.experimental import pallas as pl
from jax.experimental.pallas import tpu as pltpu
from jax.experimental.pallas import tpu_sc as plsc

N = 10000
D = 256
E = 160000
OUT = 64
H = 128
NS = 16
BE = 128
NB_ALL = E // BE
NB_LO = NB_ALL // NS
NB_EXTRA = NB_ALL - NB_LO * NS
NBT = 80
NBL = NB_ALL - NBT * (NS - 1)
ROWS_PER_TILE = 624
ROW_TAIL = N - ROWS_PER_TILE * NS

_mesh = plsc.VectorSubcoreMesh(
    core_axis_name="c", subcore_axis_name="s", num_cores=2, num_subcores=NS
)


def _leaky(v):
    return jnp.where(v >= 0, v, 0.01 * v)


@functools.partial(
    pl.kernel,
    out_type=[
        jax.ShapeDtypeStruct((N,), jnp.float32),
        jax.ShapeDtypeStruct((N,), jnp.float32),
    ],
    mesh=_mesh,
    scratch_types=[
        pltpu.VMEM((BE,), jnp.int32),
        pltpu.VMEM((BE,), jnp.float32),
        pltpu.VMEM_SHARED((N,), jnp.float32),
    ],
)
def _deg(src_hbm, dst_hbm, zer_hbm, outs_hbm, outd_hbm, idx_v, ones_v, acc_sh):
    cid = lax.axis_index("c")
    sid = lax.axis_index("s")

    def fill_ones(i, c):
        ones_v[pl.ds(i * 16, 16)] = jnp.ones((16,), jnp.float32)
        return c

    lax.fori_loop(0, BE // 16, fill_ones, 0)

    @pl.when(sid == 0)
    def _zero():
        pltpu.sync_copy(zer_hbm, acc_sh)

    plsc.subcore_barrier()

    nb = jnp.where(sid < NB_EXTRA, NB_LO + 1, NB_LO)

    def run(edges_hbm):
        def body(i, c):
            off = (sid + NS * i) * BE
            pltpu.sync_copy(edges_hbm.at[pl.ds(off, BE)], idx_v)
            pltpu.sync_copy(ones_v, acc_sh.at[idx_v], add=True)
            return c

        lax.fori_loop(0, nb, body, 0)

    @pl.when(cid == 0)
    def _src():
        run(src_hbm)

    @pl.when(cid == 1)
    def _dst():
        run(dst_hbm)

    plsc.subcore_barrier()

    @pl.when((sid == 0) & (cid == 0))
    def _ws():
        pltpu.sync_copy(acc_sh, outs_hbm)

    @pl.when((sid == 0) & (cid == 1))
    def _wd():
        pltpu.sync_copy(acc_sh, outd_hbm)


@functools.partial(
    pl.kernel,
    out_type=[
        jax.ShapeDtypeStruct((N, H), jnp.float32),
        jax.ShapeDtypeStruct((N, H), jnp.float32),
    ],
    mesh=_mesh,
    scratch_types=[
        pltpu.VMEM((2 * BE,), jnp.int32),
        pltpu.VMEM((NBT, BE), jnp.int32),
        pltpu.VMEM((2, BE, H), jnp.float32),
        pltpu.VMEM_SHARED((N, H), jnp.float32),
        pltpu.SemaphoreType.DMA,
        pltpu.SemaphoreType.DMA,
    ],
)
def _msg(t0_hbm, t1_hbm, src_hbm, dst_hbm, zrow_hbm, out0_hbm, out1_hbm,
         sidx_v, didx_v, rows_v, acc_sh, sem_g, sem_i):
    cid = lax.axis_index("c")
    sid = lax.axis_index("s")

    pltpu.sync_copy(zrow_hbm, acc_sh.at[pl.ds(sid * ROWS_PER_TILE, ROWS_PER_TILE)])

    @pl.when(sid == NS - 1)
    def _ztail():
        pltpu.sync_copy(zrow_hbm.at[pl.ds(0, ROW_TAIL)],
                        acc_sh.at[pl.ds(NS * ROWS_PER_TILE, ROW_TAIL)])

    is_last = sid == NS - 1
    row0 = sid * NBT

    @pl.when(~is_last)
    def _ld():
        pltpu.sync_copy(dst_hbm.at[pl.ds(row0, NBT)], didx_v)

    @pl.when(is_last)
    def _ldl():
        pltpu.sync_copy(dst_hbm.at[pl.ds((NS - 1) * NBT, NBL)],
                        didx_v.at[pl.ds(0, NBL)])

    plsc.subcore_barrier()

    nb = jnp.where(is_last, NBL, NBT)

    def run(t_hbm):
        pltpu.sync_copy(src_hbm.at[pl.ds(row0 * BE, BE)], sidx_v.at[pl.ds(0, BE)])

        def body(j, c):
            slot = lax.rem(j, 2)
            nxt = lax.rem(j + 1, 2)
            gcp = pltpu.async_copy(t_hbm.at[sidx_v.at[pl.ds(slot * BE, BE)]],
                                   rows_v.at[slot], sem_g)

            @pl.when(j + 1 < nb)
            def _pf():
                pltpu.async_copy(src_hbm.at[pl.ds((row0 + j + 1) * BE, BE)],
                                 sidx_v.at[pl.ds(nxt * BE, BE)], sem_i).wait()

            @pl.when(j > 0)
            def _sc():
                pltpu.sync_copy(rows_v.at[nxt], acc_sh.at[didx_v.at[j - 1]],
                                add=True)

            gcp.wait()
            return c

        lax.fori_loop(0, nb, body, 0)
        last = nb - 1
        pltpu.sync_copy(rows_v.at[lax.rem(last, 2)],
                        acc_sh.at[didx_v.at[last]], add=True)

    @pl.when(cid == 0)
    def _h0():
        run(t0_hbm)

    @pl.when(cid == 1)
    def _h1():
        run(t1_hbm)

    plsc.subcore_barrier()

    rsl = pl.ds(sid * ROWS_PER_TILE, ROWS_PER_TILE)
    tsl = pl.ds(NS * ROWS_PER_TILE, ROW_TAIL)

    @pl.when(cid == 0)
    def _w0():
        pltpu.sync_copy(acc_sh.at[rsl], out0_hbm.at[rsl])

    @pl.when(cid == 1)
    def _w1():
        pltpu.sync_copy(acc_sh.at[rsl], out1_hbm.at[rsl])

    @pl.when((sid == NS - 1) & (cid == 0))
    def _w0t():
        pltpu.sync_copy(acc_sh.at[tsl], out0_hbm.at[tsl])

    @pl.when((sid == NS - 1) & (cid == 1))
    def _w1t():
        pltpu.sync_copy(acc_sh.at[tsl], out1_hbm.at[tsl])


BM = 400


def _mm0_body(x_ref, w_ref, degs_ref, o0_ref, o1_ref):
    t = jnp.dot(x_ref[...], w_ref[...], preferred_element_type=jnp.float32)
    t = t * lax.rsqrt(degs_ref[...] + 1.0)
    o0_ref[...] = t[:, :H]
    o1_ref[...] = t[:, H:]


_mm0 = pl.pallas_call(
    _mm0_body,
    grid=(N // BM,),
    in_specs=[
        pl.BlockSpec((BM, D), lambda i: (i, 0)),
        pl.BlockSpec((D, D), lambda i: (0, 0)),
        pl.BlockSpec((BM, 1), lambda i: (i, 0)),
    ],
    out_specs=[
        pl.BlockSpec((BM, H), lambda i: (i, 0)),
        pl.BlockSpec((BM, H), lambda i: (i, 0)),
    ],
    out_shape=[
        jax.ShapeDtypeStruct((N, H), jnp.float32),
        jax.ShapeDtypeStruct((N, H), jnp.float32),
    ],
)


def _mm_body(a0_ref, a1_ref, p0_ref, p1_ref, degd_ref, b_ref, w_ref,
             degs_ref, o0_ref, o1_ref):
    agg = jnp.concatenate([a0_ref[...] + p0_ref[...],
                           a1_ref[...] + p1_ref[...]], axis=1)
    h = _leaky(agg * lax.rsqrt(degd_ref[...] + 1.0) + b_ref[...])
    t = jnp.dot(h, w_ref[...], preferred_element_type=jnp.float32)
    t = t * lax.rsqrt(degs_ref[...] + 1.0)
    o0_ref[...] = t[:, :H]
    o1_ref[...] = t[:, H:]


_mm = pl.pallas_call(
    _mm_body,
    grid=(N // BM,),
    in_specs=[
        pl.BlockSpec((BM, H), lambda i: (i, 0)),
        pl.BlockSpec((BM, H), lambda i: (i, 0)),
        pl.BlockSpec((BM, H), lambda i: (i, 0)),
        pl.BlockSpec((BM, H), lambda i: (i, 0)),
        pl.BlockSpec((BM, 1), lambda i: (i, 0)),
        pl.BlockSpec((1, D), lambda i: (0, 0)),
        pl.BlockSpec((D, D), lambda i: (0, 0)),
        pl.BlockSpec((BM, 1), lambda i: (i, 0)),
    ],
    out_specs=[
        pl.BlockSpec((BM, H), lambda i: (i, 0)),
        pl.BlockSpec((BM, H), lambda i: (i, 0)),
    ],
    out_shape=[
        jax.ShapeDtypeStruct((N, H), jnp.float32),
        jax.ShapeDtypeStruct((N, H), jnp.float32),
    ],
)


BN = 40
FC_STEPS = N // BN


def _fc_body(a0_ref, a1_ref, p0_ref, p1_ref, degd_ref, b_ref, wfc_ref,
             bfc_ref, out_ref, acc_s):
    i = pl.program_id(0)

    @pl.when(i == 0)
    def _init():
        acc_s[...] = jnp.zeros((OUT, D), jnp.float32)

    agg = jnp.concatenate([a0_ref[...] + p0_ref[...],
                           a1_ref[...] + p1_ref[...]], axis=1)
    h = _leaky(agg * lax.rsqrt(degd_ref[...] + 1.0) + b_ref[...])
    f = _leaky(h)
    w = wfc_ref[...]
    acc = acc_s[...]
    for n in range(BN):
        acc = acc + w[:, n * D:(n + 1) * D] * f[n][None, :]
    acc_s[...] = acc

    @pl.when(i == FC_STEPS - 1)
    def _fin():
        out_ref[...] = jnp.sum(acc_s[...], axis=1) + bfc_ref[...]


_fc = pl.pallas_call(
    _fc_body,
    grid=(FC_STEPS,),
    in_specs=[
        pl.BlockSpec((BN, H), lambda i: (i, 0)),
        pl.BlockSpec((BN, H), lambda i: (i, 0)),
        pl.BlockSpec((BN, H), lambda i: (i, 0)),
        pl.BlockSpec((BN, H), lambda i: (i, 0)),
        pl.BlockSpec((BN, 1), lambda i: (i, 0)),
        pl.BlockSpec((1, D), lambda i: (0, 0)),
        pl.BlockSpec((OUT, BN * D), lambda i: (0, i)),
        pl.BlockSpec((OUT,), lambda i: (0,)),
    ],
    out_specs=pl.BlockSpec((OUT,), lambda i: (0,)),
    out_shape=jax.ShapeDtypeStruct((OUT,), jnp.float32),
    scratch_shapes=[pltpu.VMEM((OUT, D), jnp.float32)],
)


def kernel(x, edge_index, W0, b0, W1, b1, W2, b2, W_fc, b_fc):
    src = edge_index[0]
    dst = edge_index[1]
    dst2 = dst.reshape(NB_ALL, BE)
    zer_deg = jnp.zeros((N,), jnp.float32)
    zer_row = jnp.zeros((ROWS_PER_TILE, H), jnp.float32)

    degs, degd = _deg(src, dst, zer_deg)
    degs = degs.reshape(N, 1)
    degd = degd.reshape(N, 1)

    t0, t1 = _mm0(x, W0, degs)
    a0, a1 = _msg(t0, t1, src, dst2, zer_row)
    t0, t1 = _mm(a0, a1, t0, t1, degd, b0.reshape(1, D), W1, degs)
    a0, a1 = _msg(t0, t1, src, dst2, zer_row)
    t0, t1 = _mm(a0, a1, t0, t1, degd, b1.reshape(1, D), W2, degs)
    a0, a1 = _msg(t0, t1, src, dst2, zer_row)
    out = _fc(a0, a1, t0, t1, degd, b2.reshape(1, D), W_fc, b_fc)
    return out


# speedup vs baseline: 7.1890x; 1.0317x over previous
import functools

import jax
import jax.numpy as jnp
from jax import lax
from jax.experimental import pallas as pl
from jax.experimental.pallas import tpu as pltpu
from jax.experimental.pallas import tpu_sc as plsc

N = 10000
D = 256
E = 160000
OUT = 64
H = 128
NS = 16
BE = 128
NB_ALL = E // BE
NB_LO = NB_ALL // NS
NB_EXTRA = NB_ALL - NB_LO * NS
NBT = 80
NBL = 48
NB_TAIL = NB_ALL - NBT * (NS - 1) - NBL
TAIL_ROW0 = NBT * (NS - 1) + NBL
ROWS_PER_TILE = 624
ROW_TAIL = N - ROWS_PER_TILE * NS

_mesh = plsc.VectorSubcoreMesh(
    core_axis_name="c", subcore_axis_name="s", num_cores=2, num_subcores=NS
)


def _leaky(v):
    return jnp.where(v >= 0, v, 0.01 * v)


@functools.partial(
    pl.kernel,
    out_type=[
        jax.ShapeDtypeStruct((N,), jnp.float32),
        jax.ShapeDtypeStruct((N,), jnp.float32),
    ],
    mesh=_mesh,
    scratch_types=[
        pltpu.VMEM((NBT, BE), jnp.int32),
        pltpu.VMEM((BE,), jnp.int32),
        pltpu.VMEM((BE,), jnp.float32),
        pltpu.VMEM_SHARED((N,), jnp.float32),
    ],
)
def _deg(ts_hbm, td_hbm, src_hbm, dst_hbm, zer_hbm, outs_hbm, outd_hbm,
         idx_v, idxt_v, ones_v, acc_sh):
    cid = lax.axis_index("c")
    sid = lax.axis_index("s")
    is_last = sid == NS - 1
    row0 = sid * NBT

    def fill_ones(i, c):
        ones_v[pl.ds(i * 16, 16)] = jnp.ones((16,), jnp.float32)
        return c

    lax.fori_loop(0, BE // 16, fill_ones, 0)

    @pl.when(sid == 0)
    def _zero():
        pltpu.sync_copy(zer_hbm, acc_sh)

    nb = jnp.where(is_last, NBL, NBT)

    def preload(edges_hbm, edges1_hbm):
        @pl.when(~is_last)
        def _ld():
            pltpu.sync_copy(edges_hbm.at[pl.ds(row0, NBT)], idx_v)

        @pl.when(is_last)
        def _ldl():
            pltpu.sync_copy(edges_hbm.at[pl.ds(row0, NBL)],
                            idx_v.at[pl.ds(0, NBL)])

        @pl.when(sid < NB_TAIL)
        def _ldt():
            pltpu.sync_copy(edges1_hbm.at[pl.ds(sid * BE, BE)], idxt_v)

    @pl.when(cid == 0)
    def _pls():
        preload(src_hbm, ts_hbm)

    @pl.when(cid == 1)
    def _pld():
        preload(dst_hbm, td_hbm)

    plsc.subcore_barrier()

    def body(i, c):
        pltpu.sync_copy(ones_v, acc_sh.at[idx_v.at[i]], add=True)
        return c

    lax.fori_loop(0, nb, body, 0)

    @pl.when(sid < NB_TAIL)
    def _tail():
        pltpu.sync_copy(ones_v, acc_sh.at[idxt_v], add=True)

    plsc.subcore_barrier()

    @pl.when((sid == 0) & (cid == 0))
    def _ws():
        pltpu.sync_copy(acc_sh, outs_hbm)

    @pl.when((sid == 0) & (cid == 1))
    def _wd():
        pltpu.sync_copy(acc_sh, outd_hbm)


@functools.partial(
    pl.kernel,
    out_type=[
        jax.ShapeDtypeStruct((N, H), jnp.float32),
        jax.ShapeDtypeStruct((N, H), jnp.float32),
    ],
    mesh=_mesh,
    scratch_types=[
        pltpu.VMEM((2 * BE,), jnp.int32),
        pltpu.VMEM((NBT, BE), jnp.int32),
        pltpu.VMEM((BE,), jnp.int32),
        pltpu.VMEM((2, BE, H), jnp.float32),
        pltpu.VMEM_SHARED((N, H), jnp.float32),
        pltpu.SemaphoreType.DMA,
        pltpu.SemaphoreType.DMA,
    ],
)
def _msg(t0_hbm, t1_hbm, src_hbm, ts_hbm, td_hbm, dst_hbm, zrow_hbm, out0_hbm,
         out1_hbm, sidx_v, didx_v, didxt_v, rows_v, acc_sh, sem_g, sem_i):
    cid = lax.axis_index("c")
    sid = lax.axis_index("s")

    pltpu.sync_copy(zrow_hbm, acc_sh.at[pl.ds(sid * ROWS_PER_TILE, ROWS_PER_TILE)])

    @pl.when(sid == NS - 1)
    def _ztail():
        pltpu.sync_copy(zrow_hbm.at[pl.ds(0, ROW_TAIL)],
                        acc_sh.at[pl.ds(NS * ROWS_PER_TILE, ROW_TAIL)])

    is_last = sid == NS - 1
    row0 = sid * NBT

    @pl.when(~is_last)
    def _ld():
        pltpu.sync_copy(dst_hbm.at[pl.ds(row0, NBT)], didx_v)

    @pl.when(is_last)
    def _ldl():
        pltpu.sync_copy(dst_hbm.at[pl.ds((NS - 1) * NBT, NBL)],
                        didx_v.at[pl.ds(0, NBL)])

    plsc.subcore_barrier()

    nb = jnp.where(is_last, NBL, NBT)

    def run(t_hbm):
        pltpu.sync_copy(src_hbm.at[pl.ds(row0 * BE, BE)], sidx_v.at[pl.ds(0, BE)])

        def body(j, c):
            slot = lax.rem(j, 2)
            nxt = lax.rem(j + 1, 2)

            @pl.when(j > 0)
            def _dr():
                pltpu.make_async_copy(src_hbm.at[pl.ds(0, BE)],
                                      sidx_v.at[pl.ds(slot * BE, BE)],
                                      sem_i).wait()

            gcp = pltpu.async_copy(t_hbm.at[sidx_v.at[pl.ds(slot * BE, BE)]],
                                   rows_v.at[slot], sem_g)

            @pl.when(j + 1 < nb)
            def _pf():
                pltpu.async_copy(src_hbm.at[pl.ds((row0 + j + 1) * BE, BE)],
                                 sidx_v.at[pl.ds(nxt * BE, BE)], sem_i)

            @pl.when(j > 0)
            def _sc():
                pltpu.sync_copy(rows_v.at[nxt], acc_sh.at[didx_v.at[j - 1]],
                                add=True)

            gcp.wait()
            return c

        lax.fori_loop(0, nb, body, 0)
        last = nb - 1
        pltpu.sync_copy(rows_v.at[lax.rem(last, 2)],
                        acc_sh.at[didx_v.at[last]], add=True)

        @pl.when(sid < NB_TAIL)
        def _tail():
            pltpu.sync_copy(ts_hbm.at[pl.ds(sid * BE, BE)],
                            sidx_v.at[pl.ds(0, BE)])
            pltpu.sync_copy(td_hbm.at[pl.ds(sid * BE, BE)], didxt_v)
            pltpu.async_copy(t_hbm.at[sidx_v.at[pl.ds(0, BE)]],
                             rows_v.at[0], sem_g).wait()
            pltpu.sync_copy(rows_v.at[0], acc_sh.at[didxt_v], add=True)

    @pl.when(cid == 0)
    def _h0():
        run(t0_hbm)

    @pl.when(cid == 1)
    def _h1():
        run(t1_hbm)

    plsc.subcore_barrier()

    rsl = pl.ds(sid * ROWS_PER_TILE, ROWS_PER_TILE)
    tsl = pl.ds(NS * ROWS_PER_TILE, ROW_TAIL)

    @pl.when(cid == 0)
    def _w0():
        pltpu.sync_copy(acc_sh.at[rsl], out0_hbm.at[rsl])

    @pl.when(cid == 1)
    def _w1():
        pltpu.sync_copy(acc_sh.at[rsl], out1_hbm.at[rsl])

    @pl.when((sid == NS - 1) & (cid == 0))
    def _w0t():
        pltpu.sync_copy(acc_sh.at[tsl], out0_hbm.at[tsl])

    @pl.when((sid == NS - 1) & (cid == 1))
    def _w1t():
        pltpu.sync_copy(acc_sh.at[tsl], out1_hbm.at[tsl])


BM = 400


def _mm0_body(x_ref, w_ref, degs_ref, o0_ref, o1_ref):
    t = jnp.dot(x_ref[...], w_ref[...], preferred_element_type=jnp.float32)
    t = t * lax.rsqrt(degs_ref[...] + 1.0)
    o0_ref[...] = t[:, :H]
    o1_ref[...] = t[:, H:]


_mm0 = pl.pallas_call(
    _mm0_body,
    grid=(N // BM,),
    in_specs=[
        pl.BlockSpec((BM, D), lambda i: (i, 0)),
        pl.BlockSpec((D, D), lambda i: (0, 0)),
        pl.BlockSpec((BM, 1), lambda i: (i, 0)),
    ],
    out_specs=[
        pl.BlockSpec((BM, H), lambda i: (i, 0)),
        pl.BlockSpec((BM, H), lambda i: (i, 0)),
    ],
    out_shape=[
        jax.ShapeDtypeStruct((N, H), jnp.float32),
        jax.ShapeDtypeStruct((N, H), jnp.float32),
    ],
)


def _mm_body(a0_ref, a1_ref, p0_ref, p1_ref, degd_ref, b_ref, w_ref,
             degs_ref, o0_ref, o1_ref):
    agg = jnp.concatenate([a0_ref[...] + p0_ref[...],
                           a1_ref[...] + p1_ref[...]], axis=1)
    h = _leaky(agg * lax.rsqrt(degd_ref[...] + 1.0) + b_ref[...])
    t = jnp.dot(h, w_ref[...], preferred_element_type=jnp.float32)
    t = t * lax.rsqrt(degs_ref[...] + 1.0)
    o0_ref[...] = t[:, :H]
    o1_ref[...] = t[:, H:]


_mm = pl.pallas_call(
    _mm_body,
    grid=(N // BM,),
    in_specs=[
        pl.BlockSpec((BM, H), lambda i: (i, 0)),
        pl.BlockSpec((BM, H), lambda i: (i, 0)),
        pl.BlockSpec((BM, H), lambda i: (i, 0)),
        pl.BlockSpec((BM, H), lambda i: (i, 0)),
        pl.BlockSpec((BM, 1), lambda i: (i, 0)),
        pl.BlockSpec((1, D), lambda i: (0, 0)),
        pl.BlockSpec((D, D), lambda i: (0, 0)),
        pl.BlockSpec((BM, 1), lambda i: (i, 0)),
    ],
    out_specs=[
        pl.BlockSpec((BM, H), lambda i: (i, 0)),
        pl.BlockSpec((BM, H), lambda i: (i, 0)),
    ],
    out_shape=[
        jax.ShapeDtypeStruct((N, H), jnp.float32),
        jax.ShapeDtypeStruct((N, H), jnp.float32),
    ],
)


BN = 40
FC_STEPS = N // BN


def _fc_body(a0_ref, a1_ref, p0_ref, p1_ref, degd_ref, b_ref, wfc_ref,
             bfc_ref, out_ref, acc_s):
    i = pl.program_id(0)

    @pl.when(i == 0)
    def _init():
        acc_s[...] = jnp.zeros((OUT, D), jnp.float32)

    agg = jnp.concatenate([a0_ref[...] + p0_ref[...],
                           a1_ref[...] + p1_ref[...]], axis=1)
    h = _leaky(agg * lax.rsqrt(degd_ref[...] + 1.0) + b_ref[...])
    f = _leaky(h)
    w = wfc_ref[...]
    acc = acc_s[...]
    for n in range(BN):
        acc = acc + w[:, n * D:(n + 1) * D] * f[n][None, :]
    acc_s[...] = acc

    @pl.when(i == FC_STEPS - 1)
    def _fin():
        out_ref[...] = jnp.sum(acc_s[...], axis=1) + bfc_ref[...]


_fc = pl.pallas_call(
    _fc_body,
    grid=(FC_STEPS,),
    in_specs=[
        pl.BlockSpec((BN, H), lambda i: (i, 0)),
        pl.BlockSpec((BN, H), lambda i: (i, 0)),
        pl.BlockSpec((BN, H), lambda i: (i, 0)),
        pl.BlockSpec((BN, H), lambda i: (i, 0)),
        pl.BlockSpec((BN, 1), lambda i: (i, 0)),
        pl.BlockSpec((1, D), lambda i: (0, 0)),
        pl.BlockSpec((OUT, BN * D), lambda i: (0, i)),
        pl.BlockSpec((OUT,), lambda i: (0,)),
    ],
    out_specs=pl.BlockSpec((OUT,), lambda i: (0,)),
    out_shape=jax.ShapeDtypeStruct((OUT,), jnp.float32),
    scratch_shapes=[pltpu.VMEM((OUT, D), jnp.float32)],
)


def kernel(x, edge_index, W0, b0, W1, b1, W2, b2, W_fc, b_fc):
    src = edge_index[0]
    dst = edge_index[1]
    src2 = src.reshape(NB_ALL, BE)
    dst2 = dst.reshape(NB_ALL, BE)
    tail_src = src[TAIL_ROW0 * BE:]
    tail_dst = dst[TAIL_ROW0 * BE:]
    zer_deg = jnp.zeros((N,), jnp.float32)
    zer_row = jnp.zeros((ROWS_PER_TILE, H), jnp.float32)

    degs, degd = _deg(tail_src, tail_dst, src2, dst2, zer_deg)
    degs = degs.reshape(N, 1)
    degd = degd.reshape(N, 1)

    t0, t1 = _mm0(x, W0, degs)
    a0, a1 = _msg(t0, t1, src, tail_src, tail_dst, dst2, zer_row)
    t0, t1 = _mm(a0, a1, t0, t1, degd, b0.reshape(1, D), W1, degs)
    a0, a1 = _msg(t0, t1, src, tail_src, tail_dst, dst2, zer_row)
    t0, t1 = _mm(a0, a1, t0, t1, degd, b1.reshape(1, D), W2, degs)
    a0, a1 = _msg(t0, t1, src, tail_src, tail_dst, dst2, zer_row)
    out = _fc(a0, a1, t0, t1, degd, b2.reshape(1, D), W_fc, b_fc)
    return out


# speedup vs baseline: 8.2923x; 1.1535x over previous
import functools

import jax
import jax.numpy as jnp
from jax import lax
from jax.experimental import pallas as pl
from jax.experimental.pallas import tpu as pltpu
from jax.experimental.pallas import tpu_sc as plsc

N = 10000
D = 256
E = 160000
OUT = 64
H = 128
NS = 16
BE = 128
NB_ALL = E // BE
NB_LO = NB_ALL // NS
NB_EXTRA = NB_ALL - NB_LO * NS
NBT = 80
NBL = 48
NB_TAIL = NB_ALL - NBT * (NS - 1) - NBL
TAIL_ROW0 = NBT * (NS - 1) + NBL
ROWS_PER_TILE = 624
ROW_TAIL = N - ROWS_PER_TILE * NS

_mesh = plsc.VectorSubcoreMesh(
    core_axis_name="c", subcore_axis_name="s", num_cores=2, num_subcores=NS
)


def _leaky(v):
    return jnp.where(v >= 0, v, 0.01 * v)


@functools.partial(
    pl.kernel,
    out_type=[
        jax.ShapeDtypeStruct((N,), jnp.float32),
        jax.ShapeDtypeStruct((N,), jnp.float32),
    ],
    mesh=_mesh,
    scratch_types=[
        pltpu.VMEM((NBT, BE), jnp.int32),
        pltpu.VMEM((BE,), jnp.int32),
        pltpu.VMEM((BE,), jnp.float32),
        pltpu.VMEM_SHARED((N,), jnp.float32),
    ],
)
def _deg(ts_hbm, td_hbm, src_hbm, dst_hbm, zer_hbm, outs_hbm, outd_hbm,
         idx_v, idxt_v, ones_v, acc_sh):
    cid = lax.axis_index("c")
    sid = lax.axis_index("s")
    is_last = sid == NS - 1
    row0 = sid * NBT

    def fill_ones(i, c):
        ones_v[pl.ds(i * 16, 16)] = jnp.ones((16,), jnp.float32)
        return c

    lax.fori_loop(0, BE // 16, fill_ones, 0)

    @pl.when(sid == 0)
    def _zero():
        pltpu.sync_copy(zer_hbm, acc_sh)

    nb = jnp.where(is_last, NBL, NBT)

    def preload(edges_hbm, edges1_hbm):
        @pl.when(~is_last)
        def _ld():
            pltpu.sync_copy(edges_hbm.at[pl.ds(row0, NBT)], idx_v)

        @pl.when(is_last)
        def _ldl():
            pltpu.sync_copy(edges_hbm.at[pl.ds(row0, NBL)],
                            idx_v.at[pl.ds(0, NBL)])

        @pl.when(sid < NB_TAIL)
        def _ldt():
            pltpu.sync_copy(edges1_hbm.at[pl.ds(sid * BE, BE)], idxt_v)

    @pl.when(cid == 0)
    def _pls():
        preload(src_hbm, ts_hbm)

    @pl.when(cid == 1)
    def _pld():
        preload(dst_hbm, td_hbm)

    plsc.subcore_barrier()

    def body(i, c):
        pltpu.sync_copy(ones_v, acc_sh.at[idx_v.at[i]], add=True)
        return c

    lax.fori_loop(0, nb, body, 0)

    @pl.when(sid < NB_TAIL)
    def _tail():
        pltpu.sync_copy(ones_v, acc_sh.at[idxt_v], add=True)

    plsc.subcore_barrier()

    @pl.when((sid == 0) & (cid == 0))
    def _ws():
        pltpu.sync_copy(acc_sh, outs_hbm)

    @pl.when((sid == 0) & (cid == 1))
    def _wd():
        pltpu.sync_copy(acc_sh, outd_hbm)


@functools.partial(
    pl.kernel,
    out_type=[
        jax.ShapeDtypeStruct((N, H), jnp.float32),
        jax.ShapeDtypeStruct((N, H), jnp.float32),
    ],
    mesh=_mesh,
    scratch_types=[
        pltpu.VMEM((4 * BE,), jnp.int32),
        pltpu.VMEM((4, BE), jnp.int32),
        pltpu.VMEM((BE,), jnp.int32),
        pltpu.VMEM((3, BE, H), jnp.float32),
        pltpu.VMEM_SHARED((N, H), jnp.float32),
        pltpu.SemaphoreType.DMA,
        pltpu.SemaphoreType.DMA,
    ],
)
def _msg(t0_hbm, t1_hbm, src_hbm, dst1_hbm, ts_hbm, td_hbm, zrow_hbm, out0_hbm,
         out1_hbm, sidx_v, didx_v, didxt_v, rows_v, acc_sh, sem_g, sem_i):
    cid = lax.axis_index("c")
    sid = lax.axis_index("s")

    pltpu.sync_copy(zrow_hbm, acc_sh.at[pl.ds(sid * ROWS_PER_TILE, ROWS_PER_TILE)])

    @pl.when(sid == NS - 1)
    def _ztail():
        pltpu.sync_copy(zrow_hbm.at[pl.ds(0, ROW_TAIL)],
                        acc_sh.at[pl.ds(NS * ROWS_PER_TILE, ROW_TAIL)])

    is_last = sid == NS - 1
    row0 = sid * NBT

    plsc.subcore_barrier()

    nb = jnp.where(is_last, NBL, NBT)

    def _ld_idx(j, slot):
        pltpu.sync_copy(src_hbm.at[pl.ds((row0 + j) * BE, BE)],
                        sidx_v.at[pl.ds(slot * BE, BE)])
        pltpu.sync_copy(dst1_hbm.at[pl.ds((row0 + j) * BE, BE)],
                        didx_v.at[slot])

    def _pf_idx(j, slot):
        pltpu.async_copy(src_hbm.at[pl.ds((row0 + j) * BE, BE)],
                         sidx_v.at[pl.ds(slot * BE, BE)], sem_i)
        pltpu.async_copy(dst1_hbm.at[pl.ds((row0 + j) * BE, BE)],
                         didx_v.at[slot], sem_i)

    def _drain_idx(slot):
        pltpu.make_async_copy(src_hbm.at[pl.ds(0, BE)],
                              sidx_v.at[pl.ds(slot * BE, BE)], sem_i).wait()
        pltpu.make_async_copy(src_hbm.at[pl.ds(0, BE)],
                              didx_v.at[slot], sem_i).wait()

    def run(t_hbm):
        _ld_idx(0, 0)
        _ld_idx(1, 1)
        pltpu.async_copy(t_hbm.at[sidx_v.at[pl.ds(0, BE)]], rows_v.at[0],
                         sem_g)
        pltpu.async_copy(t_hbm.at[sidx_v.at[pl.ds(BE, BE)]], rows_v.at[1],
                         sem_g)
        _pf_idx(2, 2)

        def body(j, c):
            @pl.when(j > 0)
            def _sc():
                pltpu.sync_copy(rows_v.at[lax.rem(j - 1, 3)],
                                acc_sh.at[didx_v.at[lax.rem(j - 1, 4)]],
                                add=True)

            @pl.when(j + 2 < nb)
            def _g2():
                s2 = lax.rem(j + 2, 4)
                _drain_idx(s2)
                pltpu.async_copy(
                    t_hbm.at[sidx_v.at[pl.ds(s2 * BE, BE)]],
                    rows_v.at[lax.rem(j + 2, 3)], sem_g)

            @pl.when(j + 3 < nb)
            def _pf():
                _pf_idx(j + 3, lax.rem(j + 3, 4))

            pltpu.make_async_copy(t_hbm.at[pl.ds(0, BE)],
                                  rows_v.at[lax.rem(j, 3)], sem_g).wait()
            return c

        lax.fori_loop(0, nb, body, 0)
        last = nb - 1
        pltpu.sync_copy(rows_v.at[lax.rem(last, 3)],
                        acc_sh.at[didx_v.at[lax.rem(last, 4)]], add=True)

        @pl.when(sid < NB_TAIL)
        def _tail():
            pltpu.sync_copy(ts_hbm.at[pl.ds(sid * BE, BE)],
                            sidx_v.at[pl.ds(0, BE)])
            pltpu.sync_copy(td_hbm.at[pl.ds(sid * BE, BE)], didxt_v)
            pltpu.async_copy(t_hbm.at[sidx_v.at[pl.ds(0, BE)]],
                             rows_v.at[0], sem_g).wait()
            pltpu.sync_copy(rows_v.at[0], acc_sh.at[didxt_v], add=True)

    @pl.when(cid == 0)
    def _h0():
        run(t0_hbm)

    @pl.when(cid == 1)
    def _h1():
        run(t1_hbm)

    plsc.subcore_barrier()

    rsl = pl.ds(sid * ROWS_PER_TILE, ROWS_PER_TILE)
    tsl = pl.ds(NS * ROWS_PER_TILE, ROW_TAIL)

    @pl.when(cid == 0)
    def _w0():
        pltpu.sync_copy(acc_sh.at[rsl], out0_hbm.at[rsl])

    @pl.when(cid == 1)
    def _w1():
        pltpu.sync_copy(acc_sh.at[rsl], out1_hbm.at[rsl])

    @pl.when((sid == NS - 1) & (cid == 0))
    def _w0t():
        pltpu.sync_copy(acc_sh.at[tsl], out0_hbm.at[tsl])

    @pl.when((sid == NS - 1) & (cid == 1))
    def _w1t():
        pltpu.sync_copy(acc_sh.at[tsl], out1_hbm.at[tsl])


BM = 400


def _mm0_body(x_ref, w_ref, degs_ref, o0_ref, o1_ref):
    t = jnp.dot(x_ref[...], w_ref[...], preferred_element_type=jnp.float32)
    t = t * lax.rsqrt(degs_ref[...] + 1.0)
    o0_ref[...] = t[:, :H]
    o1_ref[...] = t[:, H:]


_mm0 = pl.pallas_call(
    _mm0_body,
    grid=(N // BM,),
    in_specs=[
        pl.BlockSpec((BM, D), lambda i: (i, 0)),
        pl.BlockSpec((D, D), lambda i: (0, 0)),
        pl.BlockSpec((BM, 1), lambda i: (i, 0)),
    ],
    out_specs=[
        pl.BlockSpec((BM, H), lambda i: (i, 0)),
        pl.BlockSpec((BM, H), lambda i: (i, 0)),
    ],
    out_shape=[
        jax.ShapeDtypeStruct((N, H), jnp.float32),
        jax.ShapeDtypeStruct((N, H), jnp.float32),
    ],
)


def _mm_body(a0_ref, a1_ref, p0_ref, p1_ref, degd_ref, b_ref, w_ref,
             degs_ref, o0_ref, o1_ref):
    agg = jnp.concatenate([a0_ref[...] + p0_ref[...],
                           a1_ref[...] + p1_ref[...]], axis=1)
    h = _leaky(agg * lax.rsqrt(degd_ref[...] + 1.0) + b_ref[...])
    t = jnp.dot(h, w_ref[...], preferred_element_type=jnp.float32)
    t = t * lax.rsqrt(degs_ref[...] + 1.0)
    o0_ref[...] = t[:, :H]
    o1_ref[...] = t[:, H:]


_mm = pl.pallas_call(
    _mm_body,
    grid=(N // BM,),
    in_specs=[
        pl.BlockSpec((BM, H), lambda i: (i, 0)),
        pl.BlockSpec((BM, H), lambda i: (i, 0)),
        pl.BlockSpec((BM, H), lambda i: (i, 0)),
        pl.BlockSpec((BM, H), lambda i: (i, 0)),
        pl.BlockSpec((BM, 1), lambda i: (i, 0)),
        pl.BlockSpec((1, D), lambda i: (0, 0)),
        pl.BlockSpec((D, D), lambda i: (0, 0)),
        pl.BlockSpec((BM, 1), lambda i: (i, 0)),
    ],
    out_specs=[
        pl.BlockSpec((BM, H), lambda i: (i, 0)),
        pl.BlockSpec((BM, H), lambda i: (i, 0)),
    ],
    out_shape=[
        jax.ShapeDtypeStruct((N, H), jnp.float32),
        jax.ShapeDtypeStruct((N, H), jnp.float32),
    ],
)


BN = 40
FC_STEPS = N // BN


def _fc_body(a0_ref, a1_ref, p0_ref, p1_ref, degd_ref, b_ref, wfc_ref,
             bfc_ref, out_ref, acc_s):
    i = pl.program_id(0)

    @pl.when(i == 0)
    def _init():
        acc_s[...] = jnp.zeros((OUT, D), jnp.float32)

    agg = jnp.concatenate([a0_ref[...] + p0_ref[...],
                           a1_ref[...] + p1_ref[...]], axis=1)
    h = _leaky(agg * lax.rsqrt(degd_ref[...] + 1.0) + b_ref[...])
    f = _leaky(h)
    w = wfc_ref[...]
    acc = acc_s[...]
    for n in range(BN):
        acc = acc + w[:, n * D:(n + 1) * D] * f[n][None, :]
    acc_s[...] = acc

    @pl.when(i == FC_STEPS - 1)
    def _fin():
        out_ref[...] = jnp.sum(acc_s[...], axis=1) + bfc_ref[...]


_fc = pl.pallas_call(
    _fc_body,
    grid=(FC_STEPS,),
    in_specs=[
        pl.BlockSpec((BN, H), lambda i: (i, 0)),
        pl.BlockSpec((BN, H), lambda i: (i, 0)),
        pl.BlockSpec((BN, H), lambda i: (i, 0)),
        pl.BlockSpec((BN, H), lambda i: (i, 0)),
        pl.BlockSpec((BN, 1), lambda i: (i, 0)),
        pl.BlockSpec((1, D), lambda i: (0, 0)),
        pl.BlockSpec((OUT, BN * D), lambda i: (0, i)),
        pl.BlockSpec((OUT,), lambda i: (0,)),
    ],
    out_specs=pl.BlockSpec((OUT,), lambda i: (0,)),
    out_shape=jax.ShapeDtypeStruct((OUT,), jnp.float32),
    scratch_shapes=[pltpu.VMEM((OUT, D), jnp.float32)],
)


def kernel(x, edge_index, W0, b0, W1, b1, W2, b2, W_fc, b_fc):
    src = edge_index[0]
    dst = edge_index[1]
    src2 = src.reshape(NB_ALL, BE)
    dst2 = dst.reshape(NB_ALL, BE)
    tail_src = src[TAIL_ROW0 * BE:]
    tail_dst = dst[TAIL_ROW0 * BE:]
    zer_deg = jnp.zeros((N,), jnp.float32)
    zer_row = jnp.zeros((ROWS_PER_TILE, H), jnp.float32)

    degs, degd = _deg(tail_src, tail_dst, src2, dst2, zer_deg)
    degs = degs.reshape(N, 1)
    degd = degd.reshape(N, 1)

    t0, t1 = _mm0(x, W0, degs)
    a0, a1 = _msg(t0, t1, src, dst, tail_src, tail_dst, zer_row)
    t0, t1 = _mm(a0, a1, t0, t1, degd, b0.reshape(1, D), W1, degs)
    a0, a1 = _msg(t0, t1, src, dst, tail_src, tail_dst, zer_row)
    t0, t1 = _mm(a0, a1, t0, t1, degd, b1.reshape(1, D), W2, degs)
    a0, a1 = _msg(t0, t1, src, dst, tail_src, tail_dst, zer_row)
    out = _fc(a0, a1, t0, t1, degd, b2.reshape(1, D), W_fc, b_fc)
    return out
